# baseline (speedup 1.0000x reference)
import functools

import jax
import jax.numpy as jnp
from jax import lax
from jax.experimental import pallas as pl
from jax.experimental.pallas import tpu as pltpu
from jax.experimental.pallas import tpu_sc as plsc

D_MODEL = 64
SCALE = 8.0

NC = 2
NS = 16
NW = NC * NS

CR = 2
NBUF = 4


def _build(seq, rows_w, n_chunks):
    mesh = plsc.VectorSubcoreMesh(core_axis_name="c", subcore_axis_name="s")

    @functools.partial(
        pl.kernel,
        out_type=jax.ShapeDtypeStruct((rows_w * NW, D_MODEL, seq), jnp.float32),
        mesh=mesh,
        compiler_params=pltpu.CompilerParams(
            use_tc_tiling_on_sc=False, needs_layout_passes=False
        ),
        scratch_types=[
            pltpu.VMEM((rows_w, seq), jnp.int32),
            pltpu.VMEM((NBUF, CR, seq, D_MODEL), jnp.float32),
            pltpu.VMEM((NBUF, CR, D_MODEL, seq), jnp.float32),
            pltpu.SemaphoreType.DMA((NBUF,)),
            pltpu.SemaphoreType.DMA((NBUF,)),
        ],
    )
    def emb(x_hbm, tab_hbm, out_hbm, idx_v, rows_v, outb_v, gsem, osem):
        wid = lax.axis_index("s") * NC + lax.axis_index("c")
        rbase = wid * rows_w
        pltpu.sync_copy(x_hbm.at[pl.ds(rbase, rows_w)], idx_v)

        def fire_gather(g, b):
            for j in range(CR):
                pltpu.async_copy(
                    tab_hbm.at[idx_v.at[g * CR + j]],
                    rows_v.at[b, j],
                    gsem.at[b],
                )

        def drain_gather(b):
            for j in range(CR):
                pltpu.make_async_copy(
                    tab_hbm.at[idx_v.at[j]], rows_v.at[b, j], gsem.at[b]
                ).wait()

        def fire_out(g, b):
            pltpu.async_copy(
                outb_v.at[b],
                out_hbm.at[pl.ds(rbase + g * CR, CR)],
                osem.at[b],
            )

        def wait_out(b):
            pltpu.make_async_copy(
                outb_v.at[b], out_hbm.at[pl.ds(0, CR)], osem.at[b]
            ).wait()

        for g in range(NBUF - 1):
            fire_gather(g, g)

        def chunk_iter(t, carry):
            for b in range(NBUF):
                g = t * NBUF + b
                drain_gather(b)

                @pl.when(g >= NBUF)
                def _wo():
                    wait_out(b)

                def scale_row(r, c):
                    iota16 = lax.iota(jnp.int32, 16)
                    col = jnp.zeros((16,), jnp.int32) + r
                    for j in range(CR):
                        for k in range(D_MODEL // 16):
                            sl = pl.ds(16 * k, 16)
                            v = rows_v[b, j, r, sl] * SCALE
                            plsc.store_scatter(
                                outb_v.at[b, j], [iota16 + 16 * k, col], v
                            )
                    return c

                lax.fori_loop(0, seq, scale_row, 0)
                fire_out(g, b)
                nb = (b + NBUF - 1) % NBUF

                @pl.when(g + NBUF - 1 < n_chunks)
                def _prep():
                    fire_gather(g + NBUF - 1, nb)

            return carry

        lax.fori_loop(0, n_chunks // NBUF, chunk_iter, 0)
        for b in range(NBUF):
            wait_out(b)

    return emb


def kernel(x, table):
    nrows, seq = x.shape
    assert nrows % NW == 0
    rows_w = nrows // NW
    assert rows_w % CR == 0
    n_chunks = rows_w // CR
    assert n_chunks % NBUF == 0
    out = _build(seq, rows_w, n_chunks)(x.astype(jnp.int32), table)
    return out.transpose(0, 2, 1)


# speedup vs baseline: 1.7000x; 1.7000x over previous
import functools

import jax
import jax.numpy as jnp
from jax import lax
from jax.experimental import pallas as pl
from jax.experimental.pallas import tpu as pltpu
from jax.experimental.pallas import tpu_sc as plsc

D_MODEL = 64
SCALE = 8.0

NC = 2
NS = 16
NW = NC * NS

CR = 4
NBUF = 4


def _build(seq, rows_w, n_chunks):
    mesh = plsc.VectorSubcoreMesh(core_axis_name="c", subcore_axis_name="s")

    @functools.partial(
        pl.kernel,
        out_type=jax.ShapeDtypeStruct((rows_w * NW, seq, D_MODEL), jnp.float32),
        mesh=mesh,
        compiler_params=pltpu.CompilerParams(use_tc_tiling_on_sc=False),
        scratch_types=[
            pltpu.VMEM((rows_w, seq), jnp.int32),
            pltpu.VMEM((NBUF, CR, seq, D_MODEL), jnp.float32),
            pltpu.SemaphoreType.DMA((NBUF,)),
            pltpu.SemaphoreType.DMA((NBUF,)),
        ],
    )
    def emb(x_hbm, tab_hbm, out_hbm, idx_v, rows_v, gsem, osem):
        wid = lax.axis_index("s") * NC + lax.axis_index("c")
        rbase = wid * rows_w
        pltpu.sync_copy(x_hbm.at[pl.ds(rbase, rows_w)], idx_v)

        def fire_gather(g, b):
            for j in range(CR):
                pltpu.async_copy(
                    tab_hbm.at[idx_v.at[g * CR + j]],
                    rows_v.at[b, j],
                    gsem.at[b],
                )

        def drain_gather(b):
            for j in range(CR):
                pltpu.make_async_copy(
                    tab_hbm.at[idx_v.at[j]], rows_v.at[b, j], gsem.at[b]
                ).wait()

        def fire_out(g, b):
            pltpu.async_copy(
                rows_v.at[b],
                out_hbm.at[pl.ds(rbase + g * CR, CR)],
                osem.at[b],
            )

        def wait_out(b):
            pltpu.make_async_copy(
                rows_v.at[b], out_hbm.at[pl.ds(0, CR)], osem.at[b]
            ).wait()

        for g in range(NBUF - 1):
            fire_gather(g, g)

        def chunk_iter(t, carry):
            for b in range(NBUF):
                g = t * NBUF + b
                drain_gather(b)

                def scale_row(r, c):
                    for j in range(CR):
                        for k in range(D_MODEL // 16):
                            sl = pl.ds(16 * k, 16)
                            rows_v[b, j, r, sl] = rows_v[b, j, r, sl] * SCALE
                    return c

                lax.fori_loop(0, seq, scale_row, 0)
                fire_out(g, b)
                nb = (b + NBUF - 1) % NBUF

                @pl.when(g + NBUF - 1 < n_chunks)
                def _prep():
                    @pl.when(g >= 1)
                    def _w():
                        wait_out(nb)

                    fire_gather(g + NBUF - 1, nb)

            return carry

        lax.fori_loop(0, n_chunks // NBUF, chunk_iter, 0)
        for b in range(NBUF):
            wait_out(b)

    return emb


def kernel(x, table):
    nrows, seq = x.shape
    assert nrows % NW == 0
    rows_w = nrows // NW
    assert rows_w % CR == 0
    n_chunks = rows_w // CR
    assert n_chunks % NBUF == 0
    return _build(seq, rows_w, n_chunks)(x.astype(jnp.int32), table)
